# Initial kernel scaffold; baseline (speedup 1.0000x reference)
import jax
import jax.numpy as jnp
from jax.experimental import pallas as pl


def kernel(X, NX, NX_rep, W_h, gamma_h, beta_h, W_ht, gamma_ht, beta_ht, W_x, b_x, W_xt, b_xt):
    raise NotImplementedError("write your pallas kernel here")


# speedup vs baseline: 2.5156x; 2.5156x over previous
import functools

import jax
import jax.numpy as jnp
from jax import lax
from jax.experimental import pallas as pl
from jax.experimental.pallas import tpu as pltpu
from jax.experimental.pallas import tpu_sc as plsc

F = 64
NO = 68
N_TOT = 327680
N_SEG = 16384
NC = 2
NS = 16
NW = NC * NS
CHUNK = 1024
ROWS_W = N_TOT // NW
SEG_W = N_SEG // NS

_f32 = jnp.float32
_MESH = dict(core_axis_name="c", subcore_axis_name="s", num_cores=NC,
             num_subcores=NS)


def _zero_rows(buf, rows, cols):
    zv = jnp.zeros((16,), _f32)

    def body(i, _):
        for j in range(cols // 16):
            buf[i, pl.ds(j * 16, 16)] = zv
        return 0

    lax.fori_loop(0, rows, body, 0)


def _zero_flat(buf, n):
    zv = jnp.zeros((16,), _f32)

    def body(i, _):
        buf[pl.ds(i * 16, 16)] = zv
        return 0

    lax.fori_loop(0, n // 16, body, 0)


FH = F // 2


def _sc_segsum_body(x_hbm, idx_hbm, ss_out, cnt_out,
                    xbuf, idxbuf, onesbuf, zbuf, zbuf1, acc_sh, cnt_sh):
    cid = lax.axis_index("c")
    sid = lax.axis_index("s")

    _zero_rows(zbuf, SEG_W, FH)
    _zero_flat(zbuf1, SEG_W)
    ov = jnp.ones((16,), _f32)
    for r in range(8):
        for j in range(8):
            onesbuf[r, pl.ds(j * 16, 16)] = ov

    s0 = pl.multiple_of(sid * SEG_W, 8)
    pltpu.sync_copy(zbuf, acc_sh.at[pl.ds(s0, SEG_W)])
    pltpu.sync_copy(zbuf1, cnt_sh.at[pl.ds(s0, SEG_W)])
    plsc.subcore_barrier()

    rows_t = N_TOT // NS
    col0 = pl.multiple_of(cid * FH, FH)

    def chunk(ci, _):
        base = pl.multiple_of(sid * rows_t + ci * CHUNK, CHUNK)
        pltpu.sync_copy(x_hbm.at[pl.ds(base, CHUNK), pl.ds(col0, FH)], xbuf)
        crow = pl.multiple_of((sid * rows_t + ci * CHUNK) // 128, 8)
        pltpu.sync_copy(idx_hbm.at[pl.ds(crow, 8)], idxbuf)
        for j in range(8):
            pltpu.sync_copy(xbuf.at[pl.ds(j * 128, 128)],
                            acc_sh.at[idxbuf.at[j]], add=True)
            pltpu.sync_copy(onesbuf.at[j], cnt_sh.at[idxbuf.at[j]], add=True)
        return 0

    lax.fori_loop(0, rows_t // CHUNK, chunk, 0)
    plsc.subcore_barrier()

    pltpu.sync_copy(acc_sh.at[pl.ds(s0, SEG_W)],
                    ss_out.at[pl.ds(s0, SEG_W), pl.ds(col0, FH)])

    @pl.when(cid == 0)
    def _():
        pltpu.sync_copy(cnt_sh.at[pl.ds(s0, SEG_W)], cnt_out.at[pl.ds(s0, SEG_W)])


_sc_segsum = pl.kernel(
    _sc_segsum_body,
    out_type=(jax.ShapeDtypeStruct((N_SEG, F), _f32),
              jax.ShapeDtypeStruct((N_SEG,), _f32)),
    mesh=plsc.VectorSubcoreMesh(**_MESH),
    compiler_params=pltpu.CompilerParams(use_tc_tiling_on_sc=False, needs_layout_passes=False),
    scratch_types=(
        pltpu.VMEM((CHUNK, FH), _f32),
        pltpu.VMEM((8, 128), jnp.int32),
        pltpu.VMEM((8, 128), _f32),
        pltpu.VMEM((SEG_W, FH), _f32),
        pltpu.VMEM((SEG_W,), _f32),
        pltpu.VMEM_SHARED((N_SEG, FH), _f32),
        pltpu.VMEM_SHARED((N_SEG,), _f32),
    ),
)


def _sc_gather_body(tbl_hbm, idx_hbm, out_hbm, rowsbuf, idxbuf, sem):
    cid = lax.axis_index("c")
    sid = lax.axis_index("s")
    wid = sid * NC + cid

    def chunk(ci, _):
        base = pl.multiple_of(wid * ROWS_W + ci * CHUNK, CHUNK)
        crow = pl.multiple_of((wid * ROWS_W + ci * CHUNK) // 128, 8)
        pltpu.sync_copy(idx_hbm.at[pl.ds(crow, 8)], idxbuf)
        cps = [pltpu.async_copy(tbl_hbm.at[idxbuf.at[j]],
                                rowsbuf.at[pl.ds(j * 128, 128)], sem)
               for j in range(8)]
        for cp in cps:
            cp.wait()
        pltpu.sync_copy(rowsbuf, out_hbm.at[pl.ds(base, CHUNK)])
        return 0

    lax.fori_loop(0, ROWS_W // CHUNK, chunk, 0)


_sc_gather = pl.kernel(
    _sc_gather_body,
    out_type=jax.ShapeDtypeStruct((N_TOT, F), _f32),
    mesh=plsc.VectorSubcoreMesh(**_MESH),
    compiler_params=pltpu.CompilerParams(use_tc_tiling_on_sc=False, needs_layout_passes=False),
    scratch_types=(
        pltpu.VMEM((CHUNK, F), _f32),
        pltpu.VMEM((8, 128), jnp.int32),
        pltpu.SemaphoreType.DMA,
    ),
)


def _sc_finish_body(t_hbm, idx_hbm, xe_hbm, end_out, invg_out,
                    tbuf, idxbuf, outbuf, sbuf, xebuf, invbuf, endbuf,
                    zbuf1, s_sh):
    cid = lax.axis_index("c")
    sid = lax.axis_index("s")
    wid = sid * NC + cid

    _zero_flat(zbuf1, SEG_W)
    s0 = pl.multiple_of(sid * SEG_W, 8)
    pltpu.sync_copy(zbuf1, s_sh.at[pl.ds(s0, SEG_W)])
    plsc.subcore_barrier()

    rows_t = N_TOT // NS

    def chunk(ci, _):
        crow = pl.multiple_of((sid * rows_t + ci * CHUNK) // 128, 8)
        pltpu.sync_copy(t_hbm.at[pl.ds(crow, 8)], tbuf)
        pltpu.sync_copy(idx_hbm.at[pl.ds(crow, 8)], idxbuf)
        for j in range(8):
            pltpu.sync_copy(tbuf.at[j], s_sh.at[idxbuf.at[j]], add=True)
        return 0

    lax.fori_loop(0, rows_t // CHUNK, chunk, 0)
    plsc.subcore_barrier()

    pltpu.sync_copy(s_sh, sbuf)
    pltpu.sync_copy(xe_hbm, xebuf)

    def inv_body(k, _):
        sl = pl.ds(k * 16, 16)
        s = sbuf[sl]
        xe = xebuf[sl]
        v = 1.0 / (s + xe)
        invbuf[sl] = v
        endbuf[sl] = xe * v
        return 0

    lax.fori_loop(0, N_SEG // 16, inv_body, 0)

    e0 = pl.multiple_of(wid * (N_SEG // NW), 8)
    pltpu.sync_copy(endbuf.at[pl.ds(e0, N_SEG // NW)],
                    end_out.at[pl.ds(e0, N_SEG // NW)])

    def gchunk(ci, _):
        crow = pl.multiple_of((wid * ROWS_W + ci * CHUNK) // 128, 8)
        pltpu.sync_copy(idx_hbm.at[pl.ds(crow, 8)], idxbuf)
        for r in range(8):
            for c2 in range(8):
                iv = idxbuf[r, pl.ds(c2 * 16, 16)]
                outbuf[r, pl.ds(c2 * 16, 16)] = plsc.load_gather(invbuf, [iv])
        pltpu.sync_copy(outbuf, invg_out.at[pl.ds(crow, 8)])
        return 0

    lax.fori_loop(0, ROWS_W // CHUNK, gchunk, 0)


_sc_finish = pl.kernel(
    _sc_finish_body,
    out_type=(jax.ShapeDtypeStruct((N_SEG,), _f32),
              jax.ShapeDtypeStruct((N_TOT // 128, 128), _f32)),
    mesh=plsc.VectorSubcoreMesh(**_MESH),
    compiler_params=pltpu.CompilerParams(use_tc_tiling_on_sc=False, needs_layout_passes=False),
    scratch_types=(
        pltpu.VMEM((8, 128), _f32),
        pltpu.VMEM((8, 128), jnp.int32),
        pltpu.VMEM((8, 128), _f32),
        pltpu.VMEM((N_SEG,), _f32),
        pltpu.VMEM((N_SEG,), _f32),
        pltpu.VMEM((N_SEG,), _f32),
        pltpu.VMEM((N_SEG,), _f32),
        pltpu.VMEM((SEG_W,), _f32),
        pltpu.VMEM_SHARED((N_SEG,), _f32),
    ),
)


def _tc_gram_body(x_ref, g_ref):
    i = pl.program_id(0)

    @pl.when(i == 0)
    def _():
        g_ref[...] = jnp.zeros_like(g_ref)

    xb = x_ref[...]
    g_ref[...] += lax.dot_general(xb, xb, (((0,), (0,)), ((), ())),
                                  preferred_element_type=_f32)


RB_G = 2048
_tc_gram = pl.pallas_call(
    _tc_gram_body,
    grid=(N_TOT // RB_G,),
    in_specs=[pl.BlockSpec((RB_G, F), lambda i: (i, 0))],
    out_specs=pl.BlockSpec((F, F), lambda i: (0, 0)),
    out_shape=jax.ShapeDtypeStruct((F, F), _f32),
)


def _tc_small_body(ss_ref, cnt_ref, nx, g, m1, m2, mht, g_h, b_h, g_ht, b_ht,
                   mxt, bxt, b2_out, m1c_out, xe_out):
    ss = ss_ref[...]
    cnt = cnt_ref[...]
    nxf = nx[...].astype(_f32)
    x_end = ss / nxf
    m1v = m1[...]
    e2 = jnp.dot(x_end, m2[...], preferred_element_type=_f32)
    ssw = jnp.dot(ss, m1v, preferred_element_type=_f32)
    colsum_x = jnp.sum(ss, axis=0, keepdims=True)
    sum_y = (jnp.dot(colsum_x, m1v, preferred_element_type=_f32)
             + jnp.sum(cnt * e2, axis=0, keepdims=True))
    gm1 = jnp.dot(g[...], m1v, preferred_element_type=_f32)
    diag = jnp.sum(m1v * gm1, axis=0, keepdims=True)
    sum_y2 = (diag + 2.0 * jnp.sum(ssw * e2, axis=0, keepdims=True)
              + jnp.sum(cnt * e2 * e2, axis=0, keepdims=True))
    n = _f32(N_TOT)
    m = sum_y / n
    var = sum_y2 / n - m * m
    c1 = g_h[...] * lax.rsqrt(var + 1e-5)
    c0 = b_h[...] - m * c1
    b2_out[...] = e2 * c1 + c0
    m1c_out[...] = m1v * c1

    yt = jnp.dot(x_end, mht[...], preferred_element_type=_f32)
    mt = jnp.mean(yt, axis=0, keepdims=True)
    vt = jnp.mean(yt * yt, axis=0, keepdims=True) - mt * mt
    ht = jnp.maximum((yt - mt) * lax.rsqrt(vt + 1e-5) * g_ht[...] + b_ht[...],
                     0.0)
    xe_out[...] = jnp.exp(jnp.dot(ht, mxt[...], preferred_element_type=_f32)
                          + bxt[...])


_tc_small = pl.pallas_call(
    _tc_small_body,
    out_shape=(jax.ShapeDtypeStruct((N_SEG, F), _f32),
               jax.ShapeDtypeStruct((F, F), _f32),
               jax.ShapeDtypeStruct((N_SEG, 1), _f32)),
)


RB = 512


def _tc_logits(x_ref, b2g_ref, m1c_ref, mxT_ref, bx_ref):
    xh = jnp.maximum(
        jnp.dot(x_ref[...], m1c_ref[...], preferred_element_type=_f32)
        + b2g_ref[...], 0.0)
    return jnp.exp(jnp.dot(xh, mxT_ref[...], preferred_element_type=_f32)
                   + bx_ref[...])


def _tc_pass1_body(x_ref, b2g_ref, m1c_ref, mxT_ref, bx_ref, t_out):
    xx = _tc_logits(x_ref, b2g_ref, m1c_ref, mxT_ref, bx_ref)
    t_out[...] = jnp.sum(xx, axis=1, keepdims=True)


_tc_pass1 = pl.pallas_call(
    _tc_pass1_body,
    grid=(N_TOT // RB,),
    in_specs=[
        pl.BlockSpec((RB, F), lambda i: (i, 0)),
        pl.BlockSpec((RB, F), lambda i: (i, 0)),
        pl.BlockSpec((F, F), lambda i: (0, 0)),
        pl.BlockSpec((F, NO), lambda i: (0, 0)),
        pl.BlockSpec((1, NO), lambda i: (0, 0)),
    ],
    out_specs=pl.BlockSpec((RB, 1), lambda i: (i, 0)),
    out_shape=jax.ShapeDtypeStruct((N_TOT, 1), _f32),
)


def _tc_pass2_body(x_ref, b2g_ref, invg_ref, m1c_ref, mxT_ref, bx_ref,
                   app_out, con_out):
    xx = _tc_logits(x_ref, b2g_ref, m1c_ref, mxT_ref, bx_ref)
    xs = xx * invg_ref[...]
    con_out[...] = xs[:, :4]
    app_out[...] = xs[:, 4:NO]


_tc_pass2 = pl.pallas_call(
    _tc_pass2_body,
    grid=(N_TOT // RB,),
    in_specs=[
        pl.BlockSpec((RB, F), lambda i: (i, 0)),
        pl.BlockSpec((RB, F), lambda i: (i, 0)),
        pl.BlockSpec((RB, 1), lambda i: (i, 0)),
        pl.BlockSpec((F, F), lambda i: (0, 0)),
        pl.BlockSpec((F, NO), lambda i: (0, 0)),
        pl.BlockSpec((1, NO), lambda i: (0, 0)),
    ],
    out_specs=(pl.BlockSpec((RB, F), lambda i: (i, 0)),
               pl.BlockSpec((RB, 4), lambda i: (i, 0))),
    out_shape=(jax.ShapeDtypeStruct((N_TOT, F), _f32),
               jax.ShapeDtypeStruct((N_TOT, 4), _f32)),
)


def kernel(X, NX, NX_rep, W_h, gamma_h, beta_h, W_ht, gamma_ht, beta_ht,
           W_x, b_x, W_xt, b_xt):
    idx2d = NX_rep.reshape(N_TOT // 128, 128)
    m1 = W_h[:, :F].T
    m2 = W_h[:, F:].T
    mht = W_ht.T
    mxT = W_x.T
    mxt = W_xt.T

    ss, cnt = _sc_segsum(X, idx2d)
    g = _tc_gram(X)
    b2, m1c, x_end = _tc_small(
        ss, cnt.reshape(N_SEG, 1), NX.reshape(N_SEG, 1), g,
        m1, m2, mht, gamma_h.reshape(1, F), beta_h.reshape(1, F),
        gamma_ht.reshape(1, F), beta_ht.reshape(1, F), mxt,
        b_xt.reshape(1, 1))
    b2g = _sc_gather(b2, idx2d)
    t = _tc_pass1(X, b2g, m1c, mxT, b_x.reshape(1, NO))
    end, invg2d = _sc_finish(t.reshape(N_TOT // 128, 128), idx2d,
                             x_end.reshape(N_SEG))
    app, con = _tc_pass2(X, b2g, invg2d.reshape(N_TOT, 1), m1c, mxT,
                         b_x.reshape(1, NO))
    return app.reshape(N_TOT, 16, 4), con, end
